# baseline (speedup 1.0000x reference)
import functools

import jax
import jax.numpy as jnp
from jax import lax
from jax.experimental import pallas as pl
from jax.experimental.pallas import tpu as pltpu
from jax.experimental.pallas import tpu_sc as plsc

B, A, G, C = 16, 3, 76, 80
NT = 64
NZ = 4 * NT
GG = G * G
CELLS = B * A * GG
LANES = 128
ROWS = CELLS // LANES
IGNORE_THRES = 0.7
NOOBJ_SCALE = 100.0

_f32 = jnp.float32
_i32 = jnp.int32


_SC_OUT_TYPE = (
        jax.ShapeDtypeStruct((NZ, LANES), _f32),
        jax.ShapeDtypeStruct((NT, LANES), _f32),
        jax.ShapeDtypeStruct((NT, LANES), _f32),
        jax.ShapeDtypeStruct((NT, LANES), _f32),
        jax.ShapeDtypeStruct((NT, LANES), _f32),
        jax.ShapeDtypeStruct((2 * NT, LANES), _f32),
)
_SC_SCRATCH = [
        pltpu.VMEM((NT * 6,), _f32),
        pltpu.VMEM((96,), _f32),
        pltpu.VMEM((2, LANES), _i32),
        pltpu.VMEM((NT,), _i32),
        pltpu.VMEM((2 * NT,), _i32),
        pltpu.VMEM((NZ, LANES), _f32),
        pltpu.VMEM((NT, LANES), _f32),
        pltpu.VMEM((NT, LANES), _f32),
        pltpu.VMEM((NT, LANES), _f32),
        pltpu.VMEM((NT, LANES), _f32),
        pltpu.VMEM((2 * NT, LANES), _f32),
        pltpu.SemaphoreType.DMA,
]


def _sc_gather_body(t_hbm, anch_hbm, conf_hbm, cx_hbm, cy_hbm, w_hbm, h_hbm, pc_hbm,
               o_conf, o_cx, o_cy, o_w, o_h, o_pc,
               tv, av, zrow_v, crow_v, cell_v,
               bconf, bcx, bcy, bw, bh, bpc, sem):
    tile0 = (lax.axis_index("c") == 0) & (lax.axis_index("s") == 0)

    @pl.when(tile0)
    def _():
        pltpu.sync_copy(t_hbm, tv)
        pltpu.sync_copy(anch_hbm, av)

        for c in range(NT // 16):

            def col(k):
                return tv[pl.ds(k * NT + c * 16, 16)]

            si_i = col(0).astype(_i32)
            gx = col(2) * float(G)
            gy = col(3) * float(G)
            gw = col(4) * float(G)
            gh = col(5) * float(G)
            gi = gx.astype(_i32)
            gj = gy.astype(_i32)

            best = jnp.zeros((16,), _i32)
            bv = jnp.full((16,), -1.0, _f32)
            for a in range(A):
                aw = av[pl.ds((2 * a) * 16, 16)]
                ah = av[pl.ds((2 * a + 1) * 16, 16)]
                inter = jnp.minimum(aw, gw) * jnp.minimum(ah, gh)
                union = aw * ah + 1e-16 + gw * gh - inter
                iou = inter / union
                m = iou > bv
                best = jnp.where(m, a, best)
                bv = jnp.where(m, iou, bv)

            cellbase = si_i * (A * GG) + gj * G + gi
            cell = cellbase + best * GG
            pb = lax.shift_right_logical(cell * C, 7)
            cell_v[pl.ds(c * 16, 16)] = pb
            cell_v[pl.ds(NT + c * 16, 16)] = jnp.minimum(pb + 1, CELLS * C // LANES - 1)
            row = lax.shift_right_logical(cell, 7)
            crow_v[pl.ds(c * 16, 16)] = row
            k0 = c * 16
            zrow_v[k0 // LANES, pl.ds(k0 % LANES, 16)] = row
            for a in range(A):
                k = NT + a * NT + c * 16
                arow = lax.shift_right_logical(cellbase + a * GG, 7)
                zrow_v[k // LANES, pl.ds(k % LANES, 16)] = arow

        cps = [
            pltpu.async_copy(conf_hbm.at[zrow_v.at[0]], bconf.at[pl.ds(0, LANES)], sem),
            pltpu.async_copy(conf_hbm.at[zrow_v.at[1]], bconf.at[pl.ds(LANES, LANES)], sem),
            pltpu.async_copy(cx_hbm.at[crow_v], bcx, sem),
            pltpu.async_copy(cy_hbm.at[crow_v], bcy, sem),
            pltpu.async_copy(w_hbm.at[crow_v], bw, sem),
            pltpu.async_copy(h_hbm.at[crow_v], bh, sem),
            pltpu.async_copy(pc_hbm.at[cell_v], bpc, sem),
        ]
        for cp in cps:
            cp.wait()

        pltpu.sync_copy(bconf, o_conf)
        pltpu.sync_copy(bcx, o_cx)
        pltpu.sync_copy(bcy, o_cy)
        pltpu.sync_copy(bw, o_w)
        pltpu.sync_copy(bh, o_h)
        pltpu.sync_copy(bpc, o_pc)


@functools.cache
def _sc_gather():
    mesh = plsc.VectorSubcoreMesh(core_axis_name="c", subcore_axis_name="s")
    return pl.kernel(_sc_gather_body, out_type=_SC_OUT_TYPE, mesh=mesh,
                     scratch_types=_SC_SCRATCH)


def _sp(x):
    return jax.nn.softplus(x)


def _tc_body(conf_ref, tT_ref, t64_ref, anch_ref,
             zconf_ref, cxr_ref, cyr_ref, wr_ref, hr_ref, pc_ref, out_ref):
    S_all = jnp.sum(_sp(conf_ref[...]))

    anch = anch_ref[...]

    def per_target(t, cell_axis):
        if cell_axis == 0:
            sl = lambda k: t[k:k + 1, :]
        else:
            sl = lambda k: t[:, k:k + 1]
        si = sl(0).astype(_i32)
        lab = sl(1).astype(_i32)
        gx = sl(2) * float(G)
        gy = sl(3) * float(G)
        rw = sl(4)
        rh = sl(5)
        gw = rw * float(G)
        gh = rh * float(G)
        gi = gx.astype(_i32)
        gj = gy.astype(_i32)
        ious = []
        for a in range(A):
            aw = anch[0:1, 2 * a:2 * a + 1]
            ah = anch[0:1, 2 * a + 1:2 * a + 2]
            inter = jnp.minimum(aw, gw) * jnp.minimum(ah, gh)
            union = aw * ah + 1e-16 + gw * gh - inter
            ious.append(inter / union)
        best = jnp.zeros_like(si)
        bv = jnp.full_like(gx, -1.0)
        for a in range(A):
            m = ious[a] > bv
            best = jnp.where(m, a, best)
            bv = jnp.where(m, ious[a], bv)
        cellbase = si * (A * GG) + gj * G + gi
        cell = cellbase + best * GG
        return dict(si=si, lab=lab, gx=gx, gy=gy, gw=gw, gh=gh, rw=rw, rh=rh,
                    ious=ious, best=best, cellbase=cellbase, cell=cell)

    r = per_target(tT_ref[...], 0)
    c = per_target(t64_ref[...], 1)

    i_r = lax.broadcasted_iota(_i32, (NT, NT), 1)
    i_c = lax.broadcasted_iota(_i32, (NT, NT), 0)
    later = i_r > i_c
    dup = jnp.max(jnp.where((c["cell"] == r["cell"]) & later, 1.0, 0.0),
                  axis=1, keepdims=True)
    act = 1.0 - dup
    pk_r = r["cell"] * C + r["lab"]
    pk_c = c["cell"] * C + c["lab"]
    pdup = jnp.max(jnp.where((pk_c == pk_r) & later, 1.0, 0.0),
                   axis=1, keepdims=True)
    pact = 1.0 - pdup

    zc_r = jnp.concatenate(
        [r["cell"]] + [r["cellbase"] + a * GG for a in range(A)], axis=1)
    zc_c = jnp.concatenate(
        [c["cell"]] + [c["cellbase"] + a * GG for a in range(A)], axis=0)
    onef_r = jnp.ones((1, NT), _f32)
    onef_c = jnp.ones((NT, 1), _f32)
    zf_r = jnp.concatenate(
        [onef_r] + [jnp.where(r["ious"][a] > IGNORE_THRES, 1.0, 0.0) for a in range(A)],
        axis=1)
    zf_c = jnp.concatenate(
        [onef_c] + [jnp.where(c["ious"][a] > IGNORE_THRES, 1.0, 0.0) for a in range(A)],
        axis=0)
    zi_r = lax.broadcasted_iota(_i32, (NZ, NZ), 1)
    zi_c = lax.broadcasted_iota(_i32, (NZ, NZ), 0)
    dupz = jnp.max(
        jnp.where((zc_c == zc_r) & (zi_r < zi_c) & (zf_r > 0.0), 1.0, 0.0),
        axis=1, keepdims=True)
    distinct = zf_c * (1.0 - dupz)

    zlane = lax.broadcasted_iota(_i32, (NZ, LANES), 1)
    zval = jnp.sum(jnp.where(zlane == (zc_c & (LANES - 1)), zconf_ref[...], 0.0),
                   axis=1, keepdims=True)
    S_zero = jnp.sum(distinct * _sp(zval))
    n_zeroed = jnp.sum(distinct)

    lane64 = lax.broadcasted_iota(_i32, (NT, LANES), 1)
    colsel = lane64 == (c["cell"] & (LANES - 1))
    cxv = jnp.sum(jnp.where(colsel, cxr_ref[...], 0.0), axis=1, keepdims=True)
    cyv = jnp.sum(jnp.where(colsel, cyr_ref[...], 0.0), axis=1, keepdims=True)
    wv = jnp.sum(jnp.where(colsel, wr_ref[...], 0.0), axis=1, keepdims=True)
    hv = jnp.sum(jnp.where(colsel, hr_ref[...], 0.0), axis=1, keepdims=True)
    cv = lax.slice(zval, (0, 0), (NT, 1))

    tx = c["gx"] - jnp.floor(c["gx"])
    ty = c["gy"] - jnp.floor(c["gy"])
    aw_b = jnp.where(c["best"] == 0, anch[0:1, 0:1],
                     jnp.where(c["best"] == 1, anch[0:1, 2:3], anch[0:1, 4:5]))
    ah_b = jnp.where(c["best"] == 0, anch[0:1, 1:2],
                     jnp.where(c["best"] == 1, anch[0:1, 3:4], anch[0:1, 5:6]))
    tw = jnp.log(c["gw"] / aw_b + 1e-16)
    th = jnp.log(c["gh"] / ah_b + 1e-16)
    scale = 2.0 - c["rw"] * c["rh"]
    S_x = jnp.sum(act * scale * (_sp(cxv) - cxv * tx))
    S_y = jnp.sum(act * scale * (_sp(cyv) - cyv * ty))
    S_w = jnp.sum(act * 0.5 * scale * (wv - tw) ** 2)
    S_h = jnp.sum(act * 0.5 * scale * (hv - th) ** 2)
    S_co = jnp.sum(act * (_sp(cv) - cv))
    n_obj = jnp.sum(act)

    pc = pc_ref[...]
    first = lax.slice(pc, (0, 0), (NT, LANES))
    second = lax.slice(pc, (NT, 0), (2 * NT, LANES))
    off = (c["cell"] * C) & (LANES - 1)
    j = lax.broadcasted_iota(_i32, (NT, LANES), 1)
    m1 = (j >= off) & (j < off + C)
    m2 = (j + LANES) < (off + C)
    sp_win = (jnp.sum(jnp.where(m1, _sp(first), 0.0), axis=1, keepdims=True)
              + jnp.sum(jnp.where(m2, _sp(second), 0.0), axis=1, keepdims=True))
    S_csp = jnp.sum(act * sp_win)
    labpos = off + c["lab"]
    lab_win = (jnp.sum(jnp.where(j == labpos, first, 0.0), axis=1, keepdims=True)
               + jnp.sum(jnp.where(j + LANES == labpos, second, 0.0),
                         axis=1, keepdims=True))
    S_clab = jnp.sum(pact * lab_win)

    n_noobj = float(CELLS) - n_zeroed
    loss = ((S_x + S_y + S_w + S_h) / n_obj
            + S_co / n_obj
            + NOOBJ_SCALE * (S_all - S_zero) / n_noobj
            + (S_csp - S_clab) / (n_obj * float(C)))
    out_ref[...] = jnp.reshape(loss, (1, 1))


def kernel(pred_boxes, pred_cls, center_x, center_y, width, height, confidence,
           targets, anchors):
    del pred_boxes
    conf2d = confidence.reshape(ROWS, LANES)
    cx2d = center_x.reshape(ROWS, LANES)
    cy2d = center_y.reshape(ROWS, LANES)
    w2d = width.reshape(ROWS, LANES)
    h2d = height.reshape(ROWS, LANES)
    pc2d = pred_cls.reshape(CELLS * C // LANES, LANES)
    tT = targets.T
    tflat = tT.reshape(NT * 6)
    anch_rep = jnp.repeat(anchors.reshape(6), 16)
    anch8 = anchors.reshape(1, 6)

    zconf, cxr, cyr, wr, hr, pcr = _sc_gather()(
        tflat, anch_rep, conf2d, cx2d, cy2d, w2d, h2d, pc2d)

    out = pl.pallas_call(
        _tc_body,
        out_shape=jax.ShapeDtypeStruct((1, 1), _f32),
    )(conf2d, tT, targets, anch8,
      zconf, cxr, cyr, wr, hr, pcr)
    return out.reshape(())


# speedup vs baseline: 1.8291x; 1.8291x over previous
import functools

import jax
import jax.numpy as jnp
from jax import lax
from jax.experimental import pallas as pl
from jax.experimental.pallas import tpu as pltpu
from jax.experimental.pallas import tpu_sc as plsc

B, A, G, C = 16, 3, 76, 80
NT = 64
NZ = 4 * NT
GG = G * G
CELLS = B * A * GG
LANES = 128
ROWS = CELLS // LANES
IGNORE_THRES = 0.7
NOOBJ_SCALE = 100.0

_f32 = jnp.float32
_i32 = jnp.int32


_SC_OUT_TYPE = (
        jax.ShapeDtypeStruct((NZ, LANES), _f32),
        jax.ShapeDtypeStruct((NT, LANES), _f32),
        jax.ShapeDtypeStruct((NT, LANES), _f32),
        jax.ShapeDtypeStruct((NT, LANES), _f32),
        jax.ShapeDtypeStruct((NT, LANES), _f32),
        jax.ShapeDtypeStruct((4 * NT,), _i32),
)
_SC_SCRATCH = [
        pltpu.VMEM((NT * 6,), _f32),
        pltpu.VMEM((96,), _f32),
        pltpu.VMEM((2, LANES), _i32),
        pltpu.VMEM((NT,), _i32),
        pltpu.VMEM((4 * NT,), _i32),
        pltpu.VMEM((NZ, LANES), _f32),
        pltpu.VMEM((NT, LANES), _f32),
        pltpu.VMEM((NT, LANES), _f32),
        pltpu.VMEM((NT, LANES), _f32),
        pltpu.VMEM((NT, LANES), _f32),
        pltpu.SemaphoreType.DMA,
]


def _sc_gather_body(t_hbm, anch_hbm, conf_hbm, cx_hbm, cy_hbm, w_hbm, h_hbm,
               o_conf, o_cx, o_cy, o_w, o_h, o_cidx,
               tv, av, zrow_v, crow_v, cidx_v,
               bconf, bcx, bcy, bw, bh, sem):
    tile0 = (lax.axis_index("c") == 0) & (lax.axis_index("s") == 0)

    @pl.when(tile0)
    def _():
        pltpu.sync_copy(t_hbm, tv)
        pltpu.sync_copy(anch_hbm, av)

        for c in range(NT // 16):

            def col(k):
                return tv[pl.ds(k * NT + c * 16, 16)]

            si_i = col(0).astype(_i32)
            gx = col(2) * float(G)
            gy = col(3) * float(G)
            gw = col(4) * float(G)
            gh = col(5) * float(G)
            gi = gx.astype(_i32)
            gj = gy.astype(_i32)

            best = jnp.zeros((16,), _i32)
            bv = jnp.full((16,), -1.0, _f32)
            for a in range(A):
                aw = av[pl.ds((2 * a) * 16, 16)]
                ah = av[pl.ds((2 * a + 1) * 16, 16)]
                inter = jnp.minimum(aw, gw) * jnp.minimum(ah, gh)
                union = aw * ah + 1e-16 + gw * gh - inter
                iou = inter / union
                m = iou > bv
                best = jnp.where(m, a, best)
                bv = jnp.where(m, iou, bv)

            cellbase = si_i * (A * GG) + gj * G + gi
            cell = cellbase + best * GG
            cidx_v[pl.ds(c * 16, 16)] = si_i
            cidx_v[pl.ds(NT + c * 16, 16)] = best
            cidx_v[pl.ds(2 * NT + c * 16, 16)] = gj
            cidx_v[pl.ds(3 * NT + c * 16, 16)] = gi
            row = lax.shift_right_logical(cell, 7)
            crow_v[pl.ds(c * 16, 16)] = row
            k0 = c * 16
            zrow_v[k0 // LANES, pl.ds(k0 % LANES, 16)] = row
            for a in range(A):
                k = NT + a * NT + c * 16
                arow = lax.shift_right_logical(cellbase + a * GG, 7)
                zrow_v[k // LANES, pl.ds(k % LANES, 16)] = arow

        cps = [
            pltpu.async_copy(conf_hbm.at[zrow_v.at[0]], bconf.at[pl.ds(0, LANES)], sem),
            pltpu.async_copy(conf_hbm.at[zrow_v.at[1]], bconf.at[pl.ds(LANES, LANES)], sem),
            pltpu.async_copy(cx_hbm.at[crow_v], bcx, sem),
            pltpu.async_copy(cy_hbm.at[crow_v], bcy, sem),
            pltpu.async_copy(w_hbm.at[crow_v], bw, sem),
            pltpu.async_copy(h_hbm.at[crow_v], bh, sem),
        ]
        for cp in cps:
            cp.wait()

        pltpu.sync_copy(bconf, o_conf)
        pltpu.sync_copy(bcx, o_cx)
        pltpu.sync_copy(bcy, o_cy)
        pltpu.sync_copy(bw, o_w)
        pltpu.sync_copy(bh, o_h)
        pltpu.sync_copy(cidx_v, o_cidx)


@functools.cache
def _sc_gather():
    mesh = plsc.VectorSubcoreMesh(core_axis_name="c", subcore_axis_name="s")
    return pl.kernel(_sc_gather_body, out_type=_SC_OUT_TYPE, mesh=mesh,
                     scratch_types=_SC_SCRATCH)


def _sp(x):
    return jax.nn.softplus(x)


def _tc_body(conf_ref, tT_ref, t64_ref, anch_ref,
             zconf_ref, cxr_ref, cyr_ref, wr_ref, hr_ref, pc_ref, out_ref):
    S_all = jnp.sum(_sp(conf_ref[...]))

    anch = anch_ref[...]

    def per_target(t, cell_axis):
        if cell_axis == 0:
            sl = lambda k: t[k:k + 1, :]
        else:
            sl = lambda k: t[:, k:k + 1]
        si = sl(0).astype(_i32)
        lab = sl(1).astype(_i32)
        gx = sl(2) * float(G)
        gy = sl(3) * float(G)
        rw = sl(4)
        rh = sl(5)
        gw = rw * float(G)
        gh = rh * float(G)
        gi = gx.astype(_i32)
        gj = gy.astype(_i32)
        ious = []
        for a in range(A):
            aw = anch[0:1, 2 * a:2 * a + 1]
            ah = anch[0:1, 2 * a + 1:2 * a + 2]
            inter = jnp.minimum(aw, gw) * jnp.minimum(ah, gh)
            union = aw * ah + 1e-16 + gw * gh - inter
            ious.append(inter / union)
        best = jnp.zeros_like(si)
        bv = jnp.full_like(gx, -1.0)
        for a in range(A):
            m = ious[a] > bv
            best = jnp.where(m, a, best)
            bv = jnp.where(m, ious[a], bv)
        cellbase = si * (A * GG) + gj * G + gi
        cell = cellbase + best * GG
        return dict(si=si, lab=lab, gx=gx, gy=gy, gw=gw, gh=gh, rw=rw, rh=rh,
                    ious=ious, best=best, cellbase=cellbase, cell=cell)

    r = per_target(tT_ref[...], 0)
    c = per_target(t64_ref[...], 1)

    i_r = lax.broadcasted_iota(_i32, (NT, NT), 1)
    i_c = lax.broadcasted_iota(_i32, (NT, NT), 0)
    later = i_r > i_c
    dup = jnp.max(jnp.where((c["cell"] == r["cell"]) & later, 1.0, 0.0),
                  axis=1, keepdims=True)
    act = 1.0 - dup
    pk_r = r["cell"] * C + r["lab"]
    pk_c = c["cell"] * C + c["lab"]
    pdup = jnp.max(jnp.where((pk_c == pk_r) & later, 1.0, 0.0),
                   axis=1, keepdims=True)
    pact = 1.0 - pdup

    zc_r = jnp.concatenate(
        [r["cell"]] + [r["cellbase"] + a * GG for a in range(A)], axis=1)
    zc_c = jnp.concatenate(
        [c["cell"]] + [c["cellbase"] + a * GG for a in range(A)], axis=0)
    onef_r = jnp.ones((1, NT), _f32)
    onef_c = jnp.ones((NT, 1), _f32)
    zf_r = jnp.concatenate(
        [onef_r] + [jnp.where(r["ious"][a] > IGNORE_THRES, 1.0, 0.0) for a in range(A)],
        axis=1)
    zf_c = jnp.concatenate(
        [onef_c] + [jnp.where(c["ious"][a] > IGNORE_THRES, 1.0, 0.0) for a in range(A)],
        axis=0)
    zi_r = lax.broadcasted_iota(_i32, (NZ, NZ), 1)
    zi_c = lax.broadcasted_iota(_i32, (NZ, NZ), 0)
    dupz = jnp.max(
        jnp.where((zc_c == zc_r) & (zi_r < zi_c) & (zf_r > 0.0), 1.0, 0.0),
        axis=1, keepdims=True)
    distinct = zf_c * (1.0 - dupz)

    zlane = lax.broadcasted_iota(_i32, (NZ, LANES), 1)
    zval = jnp.sum(jnp.where(zlane == (zc_c & (LANES - 1)), zconf_ref[...], 0.0),
                   axis=1, keepdims=True)
    S_zero = jnp.sum(distinct * _sp(zval))
    n_zeroed = jnp.sum(distinct)

    lane64 = lax.broadcasted_iota(_i32, (NT, LANES), 1)
    colsel = lane64 == (c["cell"] & (LANES - 1))
    cxv = jnp.sum(jnp.where(colsel, cxr_ref[...], 0.0), axis=1, keepdims=True)
    cyv = jnp.sum(jnp.where(colsel, cyr_ref[...], 0.0), axis=1, keepdims=True)
    wv = jnp.sum(jnp.where(colsel, wr_ref[...], 0.0), axis=1, keepdims=True)
    hv = jnp.sum(jnp.where(colsel, hr_ref[...], 0.0), axis=1, keepdims=True)
    cv = lax.slice(zval, (0, 0), (NT, 1))

    tx = c["gx"] - jnp.floor(c["gx"])
    ty = c["gy"] - jnp.floor(c["gy"])
    aw_b = jnp.where(c["best"] == 0, anch[0:1, 0:1],
                     jnp.where(c["best"] == 1, anch[0:1, 2:3], anch[0:1, 4:5]))
    ah_b = jnp.where(c["best"] == 0, anch[0:1, 1:2],
                     jnp.where(c["best"] == 1, anch[0:1, 3:4], anch[0:1, 5:6]))
    tw = jnp.log(c["gw"] / aw_b + 1e-16)
    th = jnp.log(c["gh"] / ah_b + 1e-16)
    scale = 2.0 - c["rw"] * c["rh"]
    S_x = jnp.sum(act * scale * (_sp(cxv) - cxv * tx))
    S_y = jnp.sum(act * scale * (_sp(cyv) - cyv * ty))
    S_w = jnp.sum(act * 0.5 * scale * (wv - tw) ** 2)
    S_h = jnp.sum(act * 0.5 * scale * (hv - th) ** 2)
    S_co = jnp.sum(act * (_sp(cv) - cv))
    n_obj = jnp.sum(act)

    pc = pc_ref[...].reshape(NT, C)
    S_csp = jnp.sum(act * jnp.sum(_sp(pc), axis=1, keepdims=True))
    labsel = lax.broadcasted_iota(_i32, (NT, C), 1) == c["lab"]
    S_clab = jnp.sum(pact * jnp.sum(jnp.where(labsel, pc, 0.0), axis=1, keepdims=True))

    n_noobj = float(CELLS) - n_zeroed
    loss = ((S_x + S_y + S_w + S_h) / n_obj
            + S_co / n_obj
            + NOOBJ_SCALE * (S_all - S_zero) / n_noobj
            + (S_csp - S_clab) / (n_obj * float(C)))
    out_ref[...] = jnp.reshape(loss, (1, 1))


def _pc_gather_body(cidx_ref, pc_ref, out_ref):
    n = pl.program_id(0)
    gi = cidx_ref[3 * NT + n]
    blk = pc_ref[0, 0, 0]
    sel = lax.broadcasted_iota(_i32, (G, C), 0) == gi
    row = jnp.sum(jnp.where(sel, blk, 0.0), axis=0, keepdims=True)
    out_ref[...] = row[None]


def kernel(pred_boxes, pred_cls, center_x, center_y, width, height, confidence,
           targets, anchors):
    del pred_boxes
    conf2d = confidence.reshape(ROWS, LANES)
    cx2d = center_x.reshape(ROWS, LANES)
    cy2d = center_y.reshape(ROWS, LANES)
    w2d = width.reshape(ROWS, LANES)
    h2d = height.reshape(ROWS, LANES)
    tT = targets.T
    tflat = tT.reshape(NT * 6)
    anch_rep = jnp.repeat(anchors.reshape(6), 16)
    anch8 = anchors.reshape(1, 6)

    zconf, cxr, cyr, wr, hr, cidx = _sc_gather()(
        tflat, anch_rep, conf2d, cx2d, cy2d, w2d, h2d)

    pcr = pl.pallas_call(
        _pc_gather_body,
        grid_spec=pltpu.PrefetchScalarGridSpec(
            num_scalar_prefetch=1,
            grid=(NT,),
            in_specs=[pl.BlockSpec(
                (1, 1, 1, G, C),
                lambda n, cidx: (cidx[n], cidx[NT + n], cidx[2 * NT + n],
                                 0, 0))],
            out_specs=pl.BlockSpec((1, 1, C), lambda n, cidx: (n, 0, 0)),
        ),
        out_shape=jax.ShapeDtypeStruct((NT, 1, C), _f32),
    )(cidx, pred_cls)

    out = pl.pallas_call(
        _tc_body,
        out_shape=jax.ShapeDtypeStruct((1, 1), _f32),
    )(conf2d, tT, targets, anch8,
      zconf, cxr, cyr, wr, hr, pcr)
    return out.reshape(())


# speedup vs baseline: 3.9200x; 2.1432x over previous
import functools

import jax
import jax.numpy as jnp
from jax import lax
from jax.experimental import pallas as pl
from jax.experimental.pallas import tpu as pltpu
from jax.experimental.pallas import tpu_sc as plsc

B, A, G, C = 16, 3, 76, 80
NT = 64
NZ = 4 * NT
GG = G * G
CELLS = B * A * GG
LANES = 128
ROWS = CELLS // LANES
IGNORE_THRES = 0.7
NOOBJ_SCALE = 100.0

_f32 = jnp.float32
_i32 = jnp.int32


_SC_OUT_TYPE = (
        jax.ShapeDtypeStruct((NZ, LANES), _f32),
        jax.ShapeDtypeStruct((NT, LANES), _f32),
        jax.ShapeDtypeStruct((NT, LANES), _f32),
        jax.ShapeDtypeStruct((NT, LANES), _f32),
        jax.ShapeDtypeStruct((NT, LANES), _f32),
        jax.ShapeDtypeStruct((4 * NT,), _i32),
)
_SC_SCRATCH = [
        pltpu.VMEM((NT * 6,), _f32),
        pltpu.VMEM((96,), _f32),
        pltpu.VMEM((2, LANES), _i32),
        pltpu.VMEM((NT,), _i32),
        pltpu.VMEM((4 * NT,), _i32),
        pltpu.VMEM((NZ, LANES), _f32),
        pltpu.VMEM((NT, LANES), _f32),
        pltpu.VMEM((NT, LANES), _f32),
        pltpu.VMEM((NT, LANES), _f32),
        pltpu.VMEM((NT, LANES), _f32),
        pltpu.SemaphoreType.DMA,
]


def _sc_gather_body(t_hbm, anch_hbm, conf_hbm, cx_hbm, cy_hbm, w_hbm, h_hbm,
               o_conf, o_cx, o_cy, o_w, o_h, o_cidx,
               tv, av, zrow_v, crow_v, cidx_v,
               bconf, bcx, bcy, bw, bh, sem):
    tile0 = (lax.axis_index("c") == 0) & (lax.axis_index("s") == 0)

    @pl.when(tile0)
    def _():
        pltpu.sync_copy(t_hbm, tv)
        pltpu.sync_copy(anch_hbm, av)

        for c in range(NT // 16):

            def col(k):
                return tv[pl.ds(k * NT + c * 16, 16)]

            si_i = col(0).astype(_i32)
            gx = col(2) * float(G)
            gy = col(3) * float(G)
            gw = col(4) * float(G)
            gh = col(5) * float(G)
            gi = gx.astype(_i32)
            gj = gy.astype(_i32)

            best = jnp.zeros((16,), _i32)
            bv = jnp.full((16,), -1.0, _f32)
            for a in range(A):
                aw = av[pl.ds((2 * a) * 16, 16)]
                ah = av[pl.ds((2 * a + 1) * 16, 16)]
                inter = jnp.minimum(aw, gw) * jnp.minimum(ah, gh)
                union = aw * ah + 1e-16 + gw * gh - inter
                iou = inter / union
                m = iou > bv
                best = jnp.where(m, a, best)
                bv = jnp.where(m, iou, bv)

            cellbase = si_i * (A * GG) + gj * G + gi
            cell = cellbase + best * GG
            cidx_v[pl.ds(c * 16, 16)] = si_i
            cidx_v[pl.ds(NT + c * 16, 16)] = best
            cidx_v[pl.ds(2 * NT + c * 16, 16)] = gj
            cidx_v[pl.ds(3 * NT + c * 16, 16)] = gi
            row = lax.shift_right_logical(cell, 7)
            crow_v[pl.ds(c * 16, 16)] = row
            k0 = c * 16
            zrow_v[k0 // LANES, pl.ds(k0 % LANES, 16)] = row
            for a in range(A):
                k = NT + a * NT + c * 16
                arow = lax.shift_right_logical(cellbase + a * GG, 7)
                zrow_v[k // LANES, pl.ds(k % LANES, 16)] = arow

        cps = [
            pltpu.async_copy(conf_hbm.at[zrow_v.at[0]], bconf.at[pl.ds(0, LANES)], sem),
            pltpu.async_copy(conf_hbm.at[zrow_v.at[1]], bconf.at[pl.ds(LANES, LANES)], sem),
            pltpu.async_copy(cx_hbm.at[crow_v], bcx, sem),
            pltpu.async_copy(cy_hbm.at[crow_v], bcy, sem),
            pltpu.async_copy(w_hbm.at[crow_v], bw, sem),
            pltpu.async_copy(h_hbm.at[crow_v], bh, sem),
        ]
        for cp in cps:
            cp.wait()

        pltpu.sync_copy(bconf, o_conf)
        pltpu.sync_copy(bcx, o_cx)
        pltpu.sync_copy(bcy, o_cy)
        pltpu.sync_copy(bw, o_w)
        pltpu.sync_copy(bh, o_h)
        pltpu.sync_copy(cidx_v, o_cidx)


@functools.cache
def _sc_gather():
    mesh = plsc.VectorSubcoreMesh(core_axis_name="c", subcore_axis_name="s")
    return pl.kernel(_sc_gather_body, out_type=_SC_OUT_TYPE, mesh=mesh,
                     scratch_types=_SC_SCRATCH)


def _sp(x):
    return jax.nn.softplus(x)


def _tc_body(conf_ref, tT_ref, t64_ref, anch_ref,
             zconf_ref, cxr_ref, cyr_ref, wr_ref, hr_ref, pc_ref, out_ref):
    S_all = jnp.sum(_sp(conf_ref[...]))

    anch = anch_ref[...]

    def per_target(t, cell_axis):
        if cell_axis == 0:
            sl = lambda k: t[k:k + 1, :]
        else:
            sl = lambda k: t[:, k:k + 1]
        si = sl(0).astype(_i32)
        lab = sl(1).astype(_i32)
        gx = sl(2) * float(G)
        gy = sl(3) * float(G)
        rw = sl(4)
        rh = sl(5)
        gw = rw * float(G)
        gh = rh * float(G)
        gi = gx.astype(_i32)
        gj = gy.astype(_i32)
        ious = []
        for a in range(A):
            aw = anch[0:1, 2 * a:2 * a + 1]
            ah = anch[0:1, 2 * a + 1:2 * a + 2]
            inter = jnp.minimum(aw, gw) * jnp.minimum(ah, gh)
            union = aw * ah + 1e-16 + gw * gh - inter
            ious.append(inter / union)
        best = jnp.zeros_like(si)
        bv = jnp.full_like(gx, -1.0)
        for a in range(A):
            m = ious[a] > bv
            best = jnp.where(m, a, best)
            bv = jnp.where(m, ious[a], bv)
        cellbase = si * (A * GG) + gj * G + gi
        cell = cellbase + best * GG
        return dict(si=si, lab=lab, gx=gx, gy=gy, gw=gw, gh=gh, rw=rw, rh=rh,
                    ious=ious, best=best, cellbase=cellbase, cell=cell)

    r = per_target(tT_ref[...], 0)
    c = per_target(t64_ref[...], 1)

    i_r = lax.broadcasted_iota(_i32, (NT, NT), 1)
    i_c = lax.broadcasted_iota(_i32, (NT, NT), 0)
    later = i_r > i_c
    dup = jnp.max(jnp.where((c["cell"] == r["cell"]) & later, 1.0, 0.0),
                  axis=1, keepdims=True)
    act = 1.0 - dup
    pk_r = r["cell"] * C + r["lab"]
    pk_c = c["cell"] * C + c["lab"]
    pdup = jnp.max(jnp.where((pk_c == pk_r) & later, 1.0, 0.0),
                   axis=1, keepdims=True)
    pact = 1.0 - pdup

    zc_r = jnp.concatenate(
        [r["cell"]] + [r["cellbase"] + a * GG for a in range(A)], axis=1)
    zc_c = jnp.concatenate(
        [c["cell"]] + [c["cellbase"] + a * GG for a in range(A)], axis=0)
    onef_r = jnp.ones((1, NT), _f32)
    onef_c = jnp.ones((NT, 1), _f32)
    zf_r = jnp.concatenate(
        [onef_r] + [jnp.where(r["ious"][a] > IGNORE_THRES, 1.0, 0.0) for a in range(A)],
        axis=1)
    zf_c = jnp.concatenate(
        [onef_c] + [jnp.where(c["ious"][a] > IGNORE_THRES, 1.0, 0.0) for a in range(A)],
        axis=0)
    zi_r = lax.broadcasted_iota(_i32, (NZ, NZ), 1)
    zi_c = lax.broadcasted_iota(_i32, (NZ, NZ), 0)
    dupz = jnp.max(
        jnp.where((zc_c == zc_r) & (zi_r < zi_c) & (zf_r > 0.0), 1.0, 0.0),
        axis=1, keepdims=True)
    distinct = zf_c * (1.0 - dupz)

    zlane = lax.broadcasted_iota(_i32, (NZ, LANES), 1)
    zval = jnp.sum(jnp.where(zlane == (zc_c & (LANES - 1)), zconf_ref[...], 0.0),
                   axis=1, keepdims=True)
    S_zero = jnp.sum(distinct * _sp(zval))
    n_zeroed = jnp.sum(distinct)

    lane64 = lax.broadcasted_iota(_i32, (NT, LANES), 1)
    colsel = lane64 == (c["cell"] & (LANES - 1))
    cxv = jnp.sum(jnp.where(colsel, cxr_ref[...], 0.0), axis=1, keepdims=True)
    cyv = jnp.sum(jnp.where(colsel, cyr_ref[...], 0.0), axis=1, keepdims=True)
    wv = jnp.sum(jnp.where(colsel, wr_ref[...], 0.0), axis=1, keepdims=True)
    hv = jnp.sum(jnp.where(colsel, hr_ref[...], 0.0), axis=1, keepdims=True)
    cv = lax.slice(zval, (0, 0), (NT, 1))

    tx = c["gx"] - jnp.floor(c["gx"])
    ty = c["gy"] - jnp.floor(c["gy"])
    aw_b = jnp.where(c["best"] == 0, anch[0:1, 0:1],
                     jnp.where(c["best"] == 1, anch[0:1, 2:3], anch[0:1, 4:5]))
    ah_b = jnp.where(c["best"] == 0, anch[0:1, 1:2],
                     jnp.where(c["best"] == 1, anch[0:1, 3:4], anch[0:1, 5:6]))
    tw = jnp.log(c["gw"] / aw_b + 1e-16)
    th = jnp.log(c["gh"] / ah_b + 1e-16)
    scale = 2.0 - c["rw"] * c["rh"]
    S_x = jnp.sum(act * scale * (_sp(cxv) - cxv * tx))
    S_y = jnp.sum(act * scale * (_sp(cyv) - cyv * ty))
    S_w = jnp.sum(act * 0.5 * scale * (wv - tw) ** 2)
    S_h = jnp.sum(act * 0.5 * scale * (hv - th) ** 2)
    S_co = jnp.sum(act * (_sp(cv) - cv))
    n_obj = jnp.sum(act)

    pc = pc_ref[...].reshape(NT, C)
    S_csp = jnp.sum(act * jnp.sum(_sp(pc), axis=1, keepdims=True))
    labsel = lax.broadcasted_iota(_i32, (NT, C), 1) == c["lab"]
    S_clab = jnp.sum(pact * jnp.sum(jnp.where(labsel, pc, 0.0), axis=1, keepdims=True))

    n_noobj = float(CELLS) - n_zeroed
    loss = ((S_x + S_y + S_w + S_h) / n_obj
            + S_co / n_obj
            + NOOBJ_SCALE * (S_all - S_zero) / n_noobj
            + (S_csp - S_clab) / (n_obj * float(C)))
    out_ref[...] = jnp.reshape(loss, (1, 1))


def _pc_gather_body(cidx_ref, pc_ref, out_ref):
    n = pl.program_id(0)
    si = cidx_ref[n]
    blk = pc_ref[0, 0, 0]
    sel = lax.broadcasted_iota(_i32, (B, C), 0) == si
    row = jnp.sum(jnp.where(sel, blk, 0.0), axis=0, keepdims=True)
    out_ref[...] = row[None]


def kernel(pred_boxes, pred_cls, center_x, center_y, width, height, confidence,
           targets, anchors):
    del pred_boxes
    conf2d = confidence.reshape(ROWS, LANES)
    cx2d = center_x.reshape(ROWS, LANES)
    cy2d = center_y.reshape(ROWS, LANES)
    w2d = width.reshape(ROWS, LANES)
    h2d = height.reshape(ROWS, LANES)
    tT = targets.T
    tflat = tT.reshape(NT * 6)
    anch_rep = jnp.repeat(anchors.reshape(6), 16)
    anch8 = anchors.reshape(1, 6)

    zconf, cxr, cyr, wr, hr, cidx = _sc_gather()(
        tflat, anch_rep, conf2d, cx2d, cy2d, w2d, h2d)

    pct = jnp.transpose(pred_cls, (1, 2, 3, 0, 4))
    pcr = pl.pallas_call(
        _pc_gather_body,
        grid_spec=pltpu.PrefetchScalarGridSpec(
            num_scalar_prefetch=1,
            grid=(NT,),
            in_specs=[pl.BlockSpec(
                (1, 1, 1, B, C),
                lambda n, cidx: (cidx[NT + n], cidx[2 * NT + n],
                                 cidx[3 * NT + n], 0, 0))],
            out_specs=pl.BlockSpec((1, 1, C), lambda n, cidx: (n, 0, 0)),
        ),
        out_shape=jax.ShapeDtypeStruct((NT, 1, C), _f32),
    )(cidx, pct)

    out = pl.pallas_call(
        _tc_body,
        out_shape=jax.ShapeDtypeStruct((1, 1), _f32),
    )(conf2d, tT, targets, anch8,
      zconf, cxr, cyr, wr, hr, pcr)
    return out.reshape(())


# speedup vs baseline: 6.0401x; 1.5408x over previous
import functools

import jax
import jax.numpy as jnp
from jax import lax
from jax.experimental import pallas as pl
from jax.experimental.pallas import tpu as pltpu
from jax.experimental.pallas import tpu_sc as plsc

B, A, G, C = 16, 3, 76, 80
NT = 64
NZ = 4 * NT
GG = G * G
CELLS = B * A * GG
LANES = 128
ROWS = CELLS // LANES
IGNORE_THRES = 0.7
NOOBJ_SCALE = 100.0

_f32 = jnp.float32
_i32 = jnp.int32


_SC_OUT_TYPE = (
        jax.ShapeDtypeStruct((NZ, LANES), _f32),
        jax.ShapeDtypeStruct((4 * NT,), _i32),
)
_SC_SCRATCH = [
        pltpu.VMEM((NT * 6,), _f32),
        pltpu.VMEM((96,), _f32),
        pltpu.VMEM((2, LANES), _i32),
        pltpu.VMEM((4 * NT,), _i32),
        pltpu.VMEM((NZ, LANES), _f32),
        pltpu.SemaphoreType.DMA,
]


def _sc_gather_body(t_hbm, anch_hbm, conf_hbm,
               o_conf, o_cidx,
               tv, av, zrow_v, cidx_v,
               bconf, sem):
    tile0 = (lax.axis_index("c") == 0) & (lax.axis_index("s") == 0)

    @pl.when(tile0)
    def _():
        pltpu.sync_copy(t_hbm, tv)
        pltpu.sync_copy(anch_hbm, av)

        for c in range(NT // 16):

            def col(k):
                return tv[pl.ds(k * NT + c * 16, 16)]

            si_i = col(0).astype(_i32)
            gx = col(2) * float(G)
            gy = col(3) * float(G)
            gw = col(4) * float(G)
            gh = col(5) * float(G)
            gi = gx.astype(_i32)
            gj = gy.astype(_i32)

            best = jnp.zeros((16,), _i32)
            bv = jnp.full((16,), -1.0, _f32)
            for a in range(A):
                aw = av[pl.ds((2 * a) * 16, 16)]
                ah = av[pl.ds((2 * a + 1) * 16, 16)]
                inter = jnp.minimum(aw, gw) * jnp.minimum(ah, gh)
                union = aw * ah + 1e-16 + gw * gh - inter
                iou = inter / union
                m = iou > bv
                best = jnp.where(m, a, best)
                bv = jnp.where(m, iou, bv)

            cellbase = si_i * (A * GG) + gj * G + gi
            cell = cellbase + best * GG
            cidx_v[pl.ds(c * 16, 16)] = si_i
            cidx_v[pl.ds(NT + c * 16, 16)] = best
            cidx_v[pl.ds(2 * NT + c * 16, 16)] = gj
            cidx_v[pl.ds(3 * NT + c * 16, 16)] = gi
            row = lax.shift_right_logical(cell, 7)
            k0 = c * 16
            zrow_v[k0 // LANES, pl.ds(k0 % LANES, 16)] = row
            for a in range(A):
                k = NT + a * NT + c * 16
                arow = lax.shift_right_logical(cellbase + a * GG, 7)
                zrow_v[k // LANES, pl.ds(k % LANES, 16)] = arow

        cps = [
            pltpu.async_copy(conf_hbm.at[zrow_v.at[0]], bconf.at[pl.ds(0, LANES)], sem),
            pltpu.async_copy(conf_hbm.at[zrow_v.at[1]], bconf.at[pl.ds(LANES, LANES)], sem),
        ]
        for cp in cps:
            cp.wait()

        pltpu.sync_copy(bconf, o_conf)
        pltpu.sync_copy(cidx_v, o_cidx)


@functools.cache
def _sc_gather():
    mesh = plsc.VectorSubcoreMesh(core_axis_name="c", subcore_axis_name="s")
    return pl.kernel(_sc_gather_body, out_type=_SC_OUT_TYPE, mesh=mesh,
                     scratch_types=_SC_SCRATCH)


def _sp(x):
    return jax.nn.softplus(x)


def _tc_body(conf_ref, tT_ref, t64_ref, anch_ref,
             zconf_ref, gv_ref, pc_ref, out_ref):
    S_all = jnp.sum(_sp(conf_ref[...]))

    anch = anch_ref[...]

    def per_target(t, cell_axis):
        if cell_axis == 0:
            sl = lambda k: t[k:k + 1, :]
        else:
            sl = lambda k: t[:, k:k + 1]
        si = sl(0).astype(_i32)
        lab = sl(1).astype(_i32)
        gx = sl(2) * float(G)
        gy = sl(3) * float(G)
        rw = sl(4)
        rh = sl(5)
        gw = rw * float(G)
        gh = rh * float(G)
        gi = gx.astype(_i32)
        gj = gy.astype(_i32)
        ious = []
        for a in range(A):
            aw = anch[0:1, 2 * a:2 * a + 1]
            ah = anch[0:1, 2 * a + 1:2 * a + 2]
            inter = jnp.minimum(aw, gw) * jnp.minimum(ah, gh)
            union = aw * ah + 1e-16 + gw * gh - inter
            ious.append(inter / union)
        best = jnp.zeros_like(si)
        bv = jnp.full_like(gx, -1.0)
        for a in range(A):
            m = ious[a] > bv
            best = jnp.where(m, a, best)
            bv = jnp.where(m, ious[a], bv)
        cellbase = si * (A * GG) + gj * G + gi
        cell = cellbase + best * GG
        return dict(si=si, lab=lab, gx=gx, gy=gy, gw=gw, gh=gh, rw=rw, rh=rh,
                    ious=ious, best=best, cellbase=cellbase, cell=cell)

    r = per_target(tT_ref[...], 0)
    c = per_target(t64_ref[...], 1)

    i_r = lax.broadcasted_iota(_i32, (NT, NT), 1)
    i_c = lax.broadcasted_iota(_i32, (NT, NT), 0)
    later = i_r > i_c
    dup = jnp.max(jnp.where((c["cell"] == r["cell"]) & later, 1.0, 0.0),
                  axis=1, keepdims=True)
    act = 1.0 - dup
    pk_r = r["cell"] * C + r["lab"]
    pk_c = c["cell"] * C + c["lab"]
    pdup = jnp.max(jnp.where((pk_c == pk_r) & later, 1.0, 0.0),
                   axis=1, keepdims=True)
    pact = 1.0 - pdup

    zc_r = jnp.concatenate(
        [r["cell"]] + [r["cellbase"] + a * GG for a in range(A)], axis=1)
    zc_c = jnp.concatenate(
        [c["cell"]] + [c["cellbase"] + a * GG for a in range(A)], axis=0)
    onef_r = jnp.ones((1, NT), _f32)
    onef_c = jnp.ones((NT, 1), _f32)
    zf_r = jnp.concatenate(
        [onef_r] + [jnp.where(r["ious"][a] > IGNORE_THRES, 1.0, 0.0) for a in range(A)],
        axis=1)
    zf_c = jnp.concatenate(
        [onef_c] + [jnp.where(c["ious"][a] > IGNORE_THRES, 1.0, 0.0) for a in range(A)],
        axis=0)
    zi_r = lax.broadcasted_iota(_i32, (NZ, NZ), 1)
    zi_c = lax.broadcasted_iota(_i32, (NZ, NZ), 0)
    dupz = jnp.max(
        jnp.where((zc_c == zc_r) & (zi_r < zi_c) & (zf_r > 0.0), 1.0, 0.0),
        axis=1, keepdims=True)
    distinct = zf_c * (1.0 - dupz)

    zlane = lax.broadcasted_iota(_i32, (NZ, LANES), 1)
    zval = jnp.sum(jnp.where(zlane == (zc_c & (LANES - 1)), zconf_ref[...], 0.0),
                   axis=1, keepdims=True)
    S_zero = jnp.sum(distinct * _sp(zval))
    n_zeroed = jnp.sum(distinct)

    gv = gv_ref[...].reshape(NT, 4)
    cxv = gv[:, 0:1]
    cyv = gv[:, 1:2]
    wv = gv[:, 2:3]
    hv = gv[:, 3:4]
    cv = lax.slice(zval, (0, 0), (NT, 1))

    tx = c["gx"] - jnp.floor(c["gx"])
    ty = c["gy"] - jnp.floor(c["gy"])
    aw_b = jnp.where(c["best"] == 0, anch[0:1, 0:1],
                     jnp.where(c["best"] == 1, anch[0:1, 2:3], anch[0:1, 4:5]))
    ah_b = jnp.where(c["best"] == 0, anch[0:1, 1:2],
                     jnp.where(c["best"] == 1, anch[0:1, 3:4], anch[0:1, 5:6]))
    tw = jnp.log(c["gw"] / aw_b + 1e-16)
    th = jnp.log(c["gh"] / ah_b + 1e-16)
    scale = 2.0 - c["rw"] * c["rh"]
    S_x = jnp.sum(act * scale * (_sp(cxv) - cxv * tx))
    S_y = jnp.sum(act * scale * (_sp(cyv) - cyv * ty))
    S_w = jnp.sum(act * 0.5 * scale * (wv - tw) ** 2)
    S_h = jnp.sum(act * 0.5 * scale * (hv - th) ** 2)
    S_co = jnp.sum(act * (_sp(cv) - cv))
    n_obj = jnp.sum(act)

    pc = pc_ref[...].reshape(NT, C)
    S_csp = jnp.sum(act * jnp.sum(_sp(pc), axis=1, keepdims=True))
    labsel = lax.broadcasted_iota(_i32, (NT, C), 1) == c["lab"]
    S_clab = jnp.sum(pact * jnp.sum(jnp.where(labsel, pc, 0.0), axis=1, keepdims=True))

    n_noobj = float(CELLS) - n_zeroed
    loss = ((S_x + S_y + S_w + S_h) / n_obj
            + S_co / n_obj
            + NOOBJ_SCALE * (S_all - S_zero) / n_noobj
            + (S_csp - S_clab) / (n_obj * float(C)))
    out_ref[...] = jnp.reshape(loss, (1, 1))


_TPS = 8


def _row_gather_body(cidx_ref, *refs):
    pc_refs = refs[0:_TPS]
    grid_refs = [refs[_TPS * (1 + g):_TPS * (2 + g)] for g in range(4)]
    out_pc, out_gv = refs[5 * _TPS], refs[5 * _TPS + 1]
    t = pl.program_id(0)
    rows, gvs = [], []
    for k in range(_TPS):
        n = _TPS * t + k
        si = cidx_ref[n]
        gi = cidx_ref[3 * NT + n]
        sisub = si % 8
        pcblk = pc_refs[k][0, 0, 0]
        selb = lax.broadcasted_iota(_i32, (8, C), 0) == sisub
        rows.append(jnp.sum(jnp.where(selb, pcblk, 0.0), axis=0,
                            keepdims=True)[None])
        sel2 = ((lax.broadcasted_iota(_i32, (8, G), 0) == sisub)
                & (lax.broadcasted_iota(_i32, (8, G), 1) == gi))
        vals = []
        for g in range(4):
            blk = grid_refs[g][k][0, 0]
            vals.append(jnp.sum(jnp.where(sel2, blk, 0.0)).reshape(1, 1, 1))
        gvs.append(jnp.concatenate(vals, axis=2))
    out_pc[...] = jnp.concatenate(rows, axis=0)
    out_gv[...] = jnp.concatenate(gvs, axis=0)


def kernel(pred_boxes, pred_cls, center_x, center_y, width, height, confidence,
           targets, anchors):
    del pred_boxes
    conf2d = confidence.reshape(ROWS, LANES)
    tT = targets.T
    tflat = tT.reshape(NT * 6)
    anch_rep = jnp.repeat(anchors.reshape(6), 16)
    anch8 = anchors.reshape(1, 6)

    zconf, cidx = _sc_gather()(tflat, anch_rep, conf2d)

    pct = jnp.transpose(pred_cls, (1, 2, 3, 0, 4))
    grids_t = [jnp.transpose(v, (1, 2, 0, 3))
               for v in (center_x, center_y, width, height)]

    def pc_map(k):
        return lambda t, cidx: (cidx[NT + _TPS * t + k],
                                cidx[2 * NT + _TPS * t + k],
                                cidx[3 * NT + _TPS * t + k],
                                cidx[_TPS * t + k] // 8, 0)

    def grid_map(k):
        return lambda t, cidx: (cidx[NT + _TPS * t + k],
                                cidx[2 * NT + _TPS * t + k],
                                cidx[_TPS * t + k] // 8, 0)

    in_specs = [pl.BlockSpec((1, 1, 1, 8, C), pc_map(k)) for k in range(_TPS)]
    for _ in range(4):
        in_specs += [pl.BlockSpec((1, 1, 8, G), grid_map(k)) for k in range(_TPS)]
    pcr, gvr = pl.pallas_call(
        _row_gather_body,
        grid_spec=pltpu.PrefetchScalarGridSpec(
            num_scalar_prefetch=1,
            grid=(NT // _TPS,),
            in_specs=in_specs,
            out_specs=[
                pl.BlockSpec((_TPS, 1, C), lambda t, cidx: (t, 0, 0)),
                pl.BlockSpec((_TPS, 1, 4), lambda t, cidx: (t, 0, 0)),
            ],
        ),
        out_shape=[
            jax.ShapeDtypeStruct((NT, 1, C), _f32),
            jax.ShapeDtypeStruct((NT, 1, 4), _f32),
        ],
    )(cidx, *([pct] * _TPS),
      *[g for gt in grids_t for g in [gt] * _TPS])

    out = pl.pallas_call(
        _tc_body,
        out_shape=jax.ShapeDtypeStruct((1, 1), _f32),
    )(conf2d, tT, targets, anch8, zconf, gvr, pcr)
    return out.reshape(())


# speedup vs baseline: 6.4002x; 1.0596x over previous
import functools

import jax
import jax.numpy as jnp
from jax import lax
from jax.experimental import pallas as pl
from jax.experimental.pallas import tpu as pltpu
from jax.experimental.pallas import tpu_sc as plsc

B, A, G, C = 16, 3, 76, 80
NT = 64
NZ = 4 * NT
GG = G * G
CELLS = B * A * GG
LANES = 128
ROWS = CELLS // LANES
IGNORE_THRES = 0.7
NOOBJ_SCALE = 100.0

_f32 = jnp.float32
_i32 = jnp.int32


_SC_OUT_TYPE = (
        jax.ShapeDtypeStruct((NZ, LANES), _f32),
)
_SC_SCRATCH = [
        pltpu.VMEM((NT * 6,), _f32),
        pltpu.VMEM((96,), _f32),
        pltpu.VMEM((2, LANES), _i32),
        pltpu.VMEM((NZ, LANES), _f32),
        pltpu.SemaphoreType.DMA,
]


def _sc_gather_body(t_hbm, anch_hbm, conf_hbm,
               o_conf,
               tv, av, zrow_v,
               bconf, sem):
    tile0 = (lax.axis_index("c") == 0) & (lax.axis_index("s") == 0)

    @pl.when(tile0)
    def _():
        pltpu.sync_copy(t_hbm, tv)
        pltpu.sync_copy(anch_hbm, av)

        for c in range(NT // 16):

            def col(k):
                return tv[pl.ds(k * NT + c * 16, 16)]

            si_i = col(0).astype(_i32)
            gx = col(2) * float(G)
            gy = col(3) * float(G)
            gw = col(4) * float(G)
            gh = col(5) * float(G)
            gi = gx.astype(_i32)
            gj = gy.astype(_i32)

            best = jnp.zeros((16,), _i32)
            bv = jnp.full((16,), -1.0, _f32)
            for a in range(A):
                aw = av[pl.ds((2 * a) * 16, 16)]
                ah = av[pl.ds((2 * a + 1) * 16, 16)]
                inter = jnp.minimum(aw, gw) * jnp.minimum(ah, gh)
                union = aw * ah + 1e-16 + gw * gh - inter
                iou = inter / union
                m = iou > bv
                best = jnp.where(m, a, best)
                bv = jnp.where(m, iou, bv)

            cellbase = si_i * (A * GG) + gj * G + gi
            cell = cellbase + best * GG
            row = lax.shift_right_logical(cell, 7)
            k0 = c * 16
            zrow_v[k0 // LANES, pl.ds(k0 % LANES, 16)] = row
            for a in range(A):
                k = NT + a * NT + c * 16
                arow = lax.shift_right_logical(cellbase + a * GG, 7)
                zrow_v[k // LANES, pl.ds(k % LANES, 16)] = arow

        cps = [
            pltpu.async_copy(conf_hbm.at[zrow_v.at[0]], bconf.at[pl.ds(0, LANES)], sem),
            pltpu.async_copy(conf_hbm.at[zrow_v.at[1]], bconf.at[pl.ds(LANES, LANES)], sem),
        ]
        for cp in cps:
            cp.wait()

        pltpu.sync_copy(bconf, o_conf)


@functools.cache
def _sc_gather():
    mesh = plsc.VectorSubcoreMesh(core_axis_name="c", subcore_axis_name="s")
    return pl.kernel(_sc_gather_body, out_type=_SC_OUT_TYPE, mesh=mesh,
                     scratch_types=_SC_SCRATCH)


def _sp(x):
    return jax.nn.softplus(x)


def _tc_body(conf_ref, tT_ref, t64_ref, anch_ref,
             zconf_ref, gv_ref, pc_ref, out_ref):
    S_all = jnp.sum(_sp(conf_ref[...]))

    anch = anch_ref[...]

    def per_target(t, cell_axis):
        if cell_axis == 0:
            sl = lambda k: t[k:k + 1, :]
        else:
            sl = lambda k: t[:, k:k + 1]
        si = sl(0).astype(_i32)
        lab = sl(1).astype(_i32)
        gx = sl(2) * float(G)
        gy = sl(3) * float(G)
        rw = sl(4)
        rh = sl(5)
        gw = rw * float(G)
        gh = rh * float(G)
        gi = gx.astype(_i32)
        gj = gy.astype(_i32)
        ious = []
        for a in range(A):
            aw = anch[0:1, 2 * a:2 * a + 1]
            ah = anch[0:1, 2 * a + 1:2 * a + 2]
            inter = jnp.minimum(aw, gw) * jnp.minimum(ah, gh)
            union = aw * ah + 1e-16 + gw * gh - inter
            ious.append(inter / union)
        best = jnp.zeros_like(si)
        bv = jnp.full_like(gx, -1.0)
        for a in range(A):
            m = ious[a] > bv
            best = jnp.where(m, a, best)
            bv = jnp.where(m, ious[a], bv)
        cellbase = si * (A * GG) + gj * G + gi
        cell = cellbase + best * GG
        return dict(si=si, lab=lab, gx=gx, gy=gy, gw=gw, gh=gh, rw=rw, rh=rh,
                    ious=ious, best=best, cellbase=cellbase, cell=cell)

    r = per_target(tT_ref[...], 0)
    c = per_target(t64_ref[...], 1)

    i_r = lax.broadcasted_iota(_i32, (NT, NT), 1)
    i_c = lax.broadcasted_iota(_i32, (NT, NT), 0)
    later = i_r > i_c
    dup = jnp.max(jnp.where((c["cell"] == r["cell"]) & later, 1.0, 0.0),
                  axis=1, keepdims=True)
    act = 1.0 - dup
    pk_r = r["cell"] * C + r["lab"]
    pk_c = c["cell"] * C + c["lab"]
    pdup = jnp.max(jnp.where((pk_c == pk_r) & later, 1.0, 0.0),
                   axis=1, keepdims=True)
    pact = 1.0 - pdup

    zc_r = jnp.concatenate(
        [r["cell"]] + [r["cellbase"] + a * GG for a in range(A)], axis=1)
    zc_c = jnp.concatenate(
        [c["cell"]] + [c["cellbase"] + a * GG for a in range(A)], axis=0)
    onef_r = jnp.ones((1, NT), _f32)
    onef_c = jnp.ones((NT, 1), _f32)
    zf_r = jnp.concatenate(
        [onef_r] + [jnp.where(r["ious"][a] > IGNORE_THRES, 1.0, 0.0) for a in range(A)],
        axis=1)
    zf_c = jnp.concatenate(
        [onef_c] + [jnp.where(c["ious"][a] > IGNORE_THRES, 1.0, 0.0) for a in range(A)],
        axis=0)
    zi_r = lax.broadcasted_iota(_i32, (NZ, NZ), 1)
    zi_c = lax.broadcasted_iota(_i32, (NZ, NZ), 0)
    dupz = jnp.max(
        jnp.where((zc_c == zc_r) & (zi_r < zi_c) & (zf_r > 0.0), 1.0, 0.0),
        axis=1, keepdims=True)
    distinct = zf_c * (1.0 - dupz)

    zlane = lax.broadcasted_iota(_i32, (NZ, LANES), 1)
    zval = jnp.sum(jnp.where(zlane == (zc_c & (LANES - 1)), zconf_ref[...], 0.0),
                   axis=1, keepdims=True)
    S_zero = jnp.sum(distinct * _sp(zval))
    n_zeroed = jnp.sum(distinct)

    gv = gv_ref[...].reshape(NT, 4)
    cxv = gv[:, 0:1]
    cyv = gv[:, 1:2]
    wv = gv[:, 2:3]
    hv = gv[:, 3:4]
    cv = lax.slice(zval, (0, 0), (NT, 1))

    tx = c["gx"] - jnp.floor(c["gx"])
    ty = c["gy"] - jnp.floor(c["gy"])
    aw_b = jnp.where(c["best"] == 0, anch[0:1, 0:1],
                     jnp.where(c["best"] == 1, anch[0:1, 2:3], anch[0:1, 4:5]))
    ah_b = jnp.where(c["best"] == 0, anch[0:1, 1:2],
                     jnp.where(c["best"] == 1, anch[0:1, 3:4], anch[0:1, 5:6]))
    tw = jnp.log(c["gw"] / aw_b + 1e-16)
    th = jnp.log(c["gh"] / ah_b + 1e-16)
    scale = 2.0 - c["rw"] * c["rh"]
    S_x = jnp.sum(act * scale * (_sp(cxv) - cxv * tx))
    S_y = jnp.sum(act * scale * (_sp(cyv) - cyv * ty))
    S_w = jnp.sum(act * 0.5 * scale * (wv - tw) ** 2)
    S_h = jnp.sum(act * 0.5 * scale * (hv - th) ** 2)
    S_co = jnp.sum(act * (_sp(cv) - cv))
    n_obj = jnp.sum(act)

    pc = pc_ref[...].reshape(NT, C)
    S_csp = jnp.sum(act * jnp.sum(_sp(pc), axis=1, keepdims=True))
    labsel = lax.broadcasted_iota(_i32, (NT, C), 1) == c["lab"]
    S_clab = jnp.sum(pact * jnp.sum(jnp.where(labsel, pc, 0.0), axis=1, keepdims=True))

    n_noobj = float(CELLS) - n_zeroed
    loss = ((S_x + S_y + S_w + S_h) / n_obj
            + S_co / n_obj
            + NOOBJ_SCALE * (S_all - S_zero) / n_noobj
            + (S_csp - S_clab) / (n_obj * float(C)))
    out_ref[...] = jnp.reshape(loss, (1, 1))


def _idx_body(tT_ref, anch_ref, out_ref):
    t = tT_ref[...]
    anch = anch_ref[...]
    si = t[0:1, :].astype(_i32)
    gx = t[2:3, :] * float(G)
    gy = t[3:4, :] * float(G)
    gw = t[4:5, :] * float(G)
    gh = t[5:6, :] * float(G)
    gi = gx.astype(_i32)
    gj = gy.astype(_i32)
    best = jnp.zeros_like(si)
    bv = jnp.full_like(gx, -1.0)
    for a in range(A):
        aw = anch[0:1, 2 * a:2 * a + 1]
        ah = anch[0:1, 2 * a + 1:2 * a + 2]
        inter = jnp.minimum(aw, gw) * jnp.minimum(ah, gh)
        union = aw * ah + 1e-16 + gw * gh - inter
        iou = inter / union
        m = iou > bv
        best = jnp.where(m, a, best)
        bv = jnp.where(m, iou, bv)
    out_ref[...] = jnp.concatenate([si, best, gj, gi], axis=0)


_TPS = 8


def _row_gather_body(cidx_ref, *refs):
    pc_refs = refs[0:_TPS]
    grid_refs = [refs[_TPS * (1 + g):_TPS * (2 + g)] for g in range(4)]
    out_pc, out_gv = refs[5 * _TPS], refs[5 * _TPS + 1]
    t = pl.program_id(0)
    rows, gvs = [], []
    for k in range(_TPS):
        n = _TPS * t + k
        si = cidx_ref[0, n]
        gi = cidx_ref[3, n]
        sisub = si % 8
        pcblk = pc_refs[k][0, 0, 0]
        selb = lax.broadcasted_iota(_i32, (8, C), 0) == sisub
        rows.append(jnp.sum(jnp.where(selb, pcblk, 0.0), axis=0,
                            keepdims=True)[None])
        sel2 = ((lax.broadcasted_iota(_i32, (8, G), 0) == sisub)
                & (lax.broadcasted_iota(_i32, (8, G), 1) == gi))
        vals = []
        for g in range(4):
            blk = grid_refs[g][k][0, 0]
            vals.append(jnp.sum(jnp.where(sel2, blk, 0.0)).reshape(1, 1, 1))
        gvs.append(jnp.concatenate(vals, axis=2))
    out_pc[...] = jnp.concatenate(rows, axis=0)
    out_gv[...] = jnp.concatenate(gvs, axis=0)


def kernel(pred_boxes, pred_cls, center_x, center_y, width, height, confidence,
           targets, anchors):
    del pred_boxes
    conf2d = confidence.reshape(ROWS, LANES)
    tT = targets.T
    tflat = tT.reshape(NT * 6)
    anch_rep = jnp.repeat(anchors.reshape(6), 16)
    anch8 = anchors.reshape(1, 6)

    zconf = _sc_gather()(tflat, anch_rep, conf2d)
    if isinstance(zconf, (tuple, list)):
        zconf = zconf[0]

    cidx = pl.pallas_call(
        _idx_body,
        out_shape=jax.ShapeDtypeStruct((4, NT), _i32),
    )(tT, anch8)

    pct = jnp.transpose(pred_cls, (1, 2, 3, 0, 4))
    grids_t = [jnp.transpose(v, (1, 2, 0, 3))
               for v in (center_x, center_y, width, height)]

    def pc_map(k):
        return lambda t, cidx: (cidx[1, _TPS * t + k],
                                cidx[2, _TPS * t + k],
                                cidx[3, _TPS * t + k],
                                cidx[0, _TPS * t + k] // 8, 0)

    def grid_map(k):
        return lambda t, cidx: (cidx[1, _TPS * t + k],
                                cidx[2, _TPS * t + k],
                                cidx[0, _TPS * t + k] // 8, 0)

    in_specs = [pl.BlockSpec((1, 1, 1, 8, C), pc_map(k)) for k in range(_TPS)]
    for _ in range(4):
        in_specs += [pl.BlockSpec((1, 1, 8, G), grid_map(k)) for k in range(_TPS)]
    pcr, gvr = pl.pallas_call(
        _row_gather_body,
        grid_spec=pltpu.PrefetchScalarGridSpec(
            num_scalar_prefetch=1,
            grid=(NT // _TPS,),
            in_specs=in_specs,
            out_specs=[
                pl.BlockSpec((_TPS, 1, C), lambda t, cidx: (t, 0, 0)),
                pl.BlockSpec((_TPS, 1, 4), lambda t, cidx: (t, 0, 0)),
            ],
        ),
        out_shape=[
            jax.ShapeDtypeStruct((NT, 1, C), _f32),
            jax.ShapeDtypeStruct((NT, 1, 4), _f32),
        ],
    )(cidx, *([pct] * _TPS),
      *[g for gt in grids_t for g in [gt] * _TPS])

    out = pl.pallas_call(
        _tc_body,
        out_shape=jax.ShapeDtypeStruct((1, 1), _f32),
    )(conf2d, tT, targets, anch8, zconf, gvr, pcr)
    return out.reshape(())
